# baseline (speedup 1.0000x reference)
import jax
import jax.numpy as jnp
from jax import lax
from jax.experimental import pallas as pl
from jax.experimental.pallas import tpu as pltpu
from jax.experimental.pallas import tpu_sc as plsc

ID_DIM = 128
LLM_DIM = 2048
OUT_DIM = 1024
E = 8
TOK = 2048
IN_DIM = ID_DIM + LLM_DIM
HID = 2 * OUT_DIM

NC = 2
NS = 16
L = 16

BR = 128
NB = TOK * 2 // BR + E
NBP = 48
R = NB * BR
RB = 1024
NBR = R // RB

_SC_MESH = dict(core_axis_name="c", subcore_axis_name="s",
                num_cores=NC, num_subcores=NS)


def _gate_body(x_ref, wg1_ref, bg1_ref, wg2_ref, bg2_ref,
               d1_ref, d2_ref, p1_ref, p2_ref, be_ref, ba_ref):
    x = x_ref[...]
    h = jnp.maximum(
        jnp.dot(x, wg1_ref[...], preferred_element_type=jnp.float32)
        + bg1_ref[...], 0.0)
    logits = (jnp.dot(h, wg2_ref[...], preferred_element_type=jnp.float32)
              + bg2_ref[...])
    iota_e = lax.broadcasted_iota(jnp.int32, (TOK, E), 1)
    m1 = jnp.max(logits, axis=1, keepdims=True)
    idx1 = jnp.min(jnp.where(logits == m1, iota_e, E), axis=1, keepdims=True)
    masked = jnp.where(iota_e == idx1, -jnp.inf, logits)
    m2 = jnp.max(masked, axis=1, keepdims=True)
    idx2 = jnp.min(jnp.where(masked == m2, iota_e, E), axis=1, keepdims=True)
    p2 = 1.0 / (1.0 + jnp.exp(m1 - m2))
    p1_ref[...] = 1.0 - p2
    p2_ref[...] = p2

    oh1 = (iota_e == idx1).astype(jnp.float32)
    oh2 = (iota_e == idx2).astype(jnp.float32)
    cnt = oh1 + oh2
    rowi = lax.broadcasted_iota(jnp.int32, (TOK, TOK), 0)
    coli = lax.broadcasted_iota(jnp.int32, (TOK, TOK), 1)
    lt = (coli < rowi).astype(jnp.float32)
    prefix = jnp.dot(lt, cnt, preferred_element_type=jnp.float32)
    tot = jnp.sum(cnt, axis=0, keepdims=True)
    padded = jnp.ceil(tot / BR) * BR
    e_r = lax.broadcasted_iota(jnp.int32, (E, E), 0)
    e_c = lax.broadcasted_iota(jnp.int32, (E, E), 1)
    lt8 = (e_r < e_c).astype(jnp.float32)
    off = jnp.dot(padded, lt8, preferred_element_type=jnp.float32)
    dest1 = jnp.sum(oh1 * (off + prefix), axis=1, keepdims=True)
    dest2 = jnp.sum(oh2 * (off + prefix + oh1), axis=1, keepdims=True)
    d1_ref[...] = dest1.astype(jnp.int32)
    d2_ref[...] = dest2.astype(jnp.int32)

    blkoff = (off / BR).reshape(E, 1)
    nblk = (padded / BR).reshape(E, 1)
    totb = jnp.sum(padded / BR)
    bv = lax.broadcasted_iota(jnp.int32, (E, NBP), 1).astype(jnp.float32)
    ev = lax.broadcasted_iota(jnp.int32, (E, NBP), 0).astype(jnp.float32)
    cand = jnp.where((bv >= blkoff) & (nblk > 0), ev, 0.0)
    be_ref[...] = jnp.max(cand, axis=0, keepdims=True).astype(jnp.int32)
    bav = lax.broadcasted_iota(jnp.int32, (1, NBP), 1).astype(jnp.float32)
    ba_ref[...] = jnp.where(bav < totb, 1, 0).astype(jnp.int32)


def _gate_route(combined, Wg1, bg1, Wg2, bg2):
    return pl.pallas_call(
        _gate_body,
        out_shape=(
            jax.ShapeDtypeStruct((TOK, 1), jnp.int32),
            jax.ShapeDtypeStruct((TOK, 1), jnp.int32),
            jax.ShapeDtypeStruct((TOK, 1), jnp.float32),
            jax.ShapeDtypeStruct((TOK, 1), jnp.float32),
            jax.ShapeDtypeStruct((1, NBP), jnp.int32),
            jax.ShapeDtypeStruct((1, NBP), jnp.int32),
        ),
        compiler_params=pltpu.CompilerParams(
            vmem_limit_bytes=100 * 1024 * 1024),
    )(combined, Wg1, bg1, Wg2, bg2)


def _invert_body(d1_ref, d2_ref, w1_ref, w2_ref, rtok_ref, rwt_ref):
    i = pl.program_id(0)
    rbase = (i * RB).astype(jnp.float32)
    colv = (lax.broadcasted_iota(jnp.int32, (TOK, RB), 1).astype(jnp.float32)
            + rbase)
    m1 = (d1_ref[...] == colv).astype(jnp.float32)
    m2 = (d2_ref[...] == colv).astype(jnp.float32)
    tid = lax.broadcasted_iota(jnp.int32, (TOK, 1), 0).astype(jnp.float32)
    rtok = jnp.sum((m1 + m2) * tid, axis=0, keepdims=True)
    rwt = jnp.sum(m1 * w1_ref[...] + m2 * w2_ref[...], axis=0, keepdims=True)
    rtok_ref[...] = rtok.astype(jnp.int32).reshape(1, 1, RB)
    rwt_ref[...] = rwt.reshape(1, 1, RB)


def _invert(d1f, d2f, w1g, w2g):
    full = pl.BlockSpec((TOK, 1), lambda i: (0, 0))
    return pl.pallas_call(
        _invert_body,
        grid=(NBR,),
        in_specs=[full, full, full, full],
        out_specs=(pl.BlockSpec((1, 1, RB), lambda i: (i, 0, 0)),
                   pl.BlockSpec((1, 1, RB), lambda i: (i, 0, 0))),
        out_shape=(jax.ShapeDtypeStruct((NBR, 1, RB), jnp.int32),
                   jax.ShapeDtypeStruct((NBR, 1, RB), jnp.float32)),
        compiler_params=pltpu.CompilerParams(
            vmem_limit_bytes=100 * 1024 * 1024),
    )(d1f, d2f, w1g, w2g)


def _gather_body(tok_hbm, comb_hbm, xs_hbm, idxv, rows, sem):
    cid = lax.axis_index("c")
    sid = lax.axis_index("s")
    wid = sid * NC + cid
    RPW = R // (NC * NS)
    CH = 32
    for ch in range(RPW // CH):
        base = wid * RPW + ch * CH
        pltpu.sync_copy(tok_hbm.at[pl.ds(base, CH)], idxv)
        pltpu.async_copy(comb_hbm.at[idxv], rows, sem).wait()
        pltpu.sync_copy(rows, xs_hbm.at[pl.ds(base, CH)])


def _gather_rows(row_token, combined):
    return pl.kernel(
        _gather_body,
        out_type=jax.ShapeDtypeStruct((R, IN_DIM), jnp.float32),
        mesh=plsc.VectorSubcoreMesh(**_SC_MESH),
        scratch_types=[
            pltpu.VMEM((32,), jnp.int32),
            pltpu.VMEM((32, IN_DIM), jnp.float32),
            pltpu.SemaphoreType.DMA,
        ],
    )(row_token, combined)


def _mlp_body(be_ref, ba_ref, xs_ref, w1_ref, b1_ref, w2_ref, b2_ref,
              wrow_ref, y_ref):
    i = pl.program_id(0)

    @pl.when(ba_ref[i] == 1)
    def _compute():
        h = jnp.maximum(
            jnp.dot(xs_ref[...], w1_ref[0],
                    preferred_element_type=jnp.float32) + b1_ref[0], 0.0)
        y = (jnp.dot(h, w2_ref[0], preferred_element_type=jnp.float32)
             + b2_ref[0])
        y_ref[...] = y * wrow_ref[0]


def _grouped_mlp(blk_e, blk_a, xs, W1, b1, W2, b2, wrow):
    grid_spec = pltpu.PrefetchScalarGridSpec(
        num_scalar_prefetch=2,
        grid=(NB,),
        in_specs=[
            pl.BlockSpec((BR, IN_DIM),
                         lambda i, be, ba: (jnp.where(ba[i] == 1, i, 0), 0)),
            pl.BlockSpec((1, IN_DIM, HID), lambda i, be, ba: (be[i], 0, 0)),
            pl.BlockSpec((1, 1, HID), lambda i, be, ba: (be[i], 0, 0)),
            pl.BlockSpec((1, HID, OUT_DIM), lambda i, be, ba: (be[i], 0, 0)),
            pl.BlockSpec((1, 1, OUT_DIM), lambda i, be, ba: (be[i], 0, 0)),
            pl.BlockSpec((1, BR, 1), lambda i, be, ba: (i, 0, 0)),
        ],
        out_specs=pl.BlockSpec((BR, OUT_DIM), lambda i, be, ba: (i, 0)),
    )
    return pl.pallas_call(
        _mlp_body,
        grid_spec=grid_spec,
        out_shape=jax.ShapeDtypeStruct((R, OUT_DIM), jnp.float32),
        compiler_params=pltpu.CompilerParams(
            vmem_limit_bytes=100 * 1024 * 1024),
    )(blk_e, blk_a, xs, W1, b1.reshape(E, 1, HID), W2,
      b2.reshape(E, 1, OUT_DIM), wrow)


def _combine_body(d1_hbm, d2_hbm, y_hbm, ya_hbm, yb_hbm, i0, i1, r0, r1, sem):
    cid = lax.axis_index("c")
    sid = lax.axis_index("s")
    wid = sid * NC + cid
    TPW = TOK // (NC * NS)
    CH = 32
    for ch in range(TPW // CH):
        base = wid * TPW + ch * CH
        pltpu.sync_copy(d1_hbm.at[pl.ds(base, CH)], i0)
        pltpu.sync_copy(d2_hbm.at[pl.ds(base, CH)], i1)
        pltpu.async_copy(y_hbm.at[i0], r0, sem).wait()
        pltpu.sync_copy(r0, ya_hbm.at[pl.ds(base, CH)])
        pltpu.async_copy(y_hbm.at[i1], r1, sem).wait()
        pltpu.sync_copy(r1, yb_hbm.at[pl.ds(base, CH)])


def _combine(d1, d2, y):
    return pl.kernel(
        _combine_body,
        out_type=(jax.ShapeDtypeStruct((TOK, OUT_DIM), jnp.float32),
                  jax.ShapeDtypeStruct((TOK, OUT_DIM), jnp.float32)),
        mesh=plsc.VectorSubcoreMesh(**_SC_MESH),
        scratch_types=[
            pltpu.VMEM((32,), jnp.int32),
            pltpu.VMEM((32,), jnp.int32),
            pltpu.VMEM((32, OUT_DIM), jnp.float32),
            pltpu.VMEM((32, OUT_DIM), jnp.float32),
            pltpu.SemaphoreType.DMA,
        ],
    )(d1, d2, y)


def _add_body(a_ref, b_ref, o_ref):
    o_ref[...] = a_ref[...] + b_ref[...]


def _pairwise_add(ya, yb):
    blk = pl.BlockSpec((256, OUT_DIM), lambda i: (i, 0))
    return pl.pallas_call(
        _add_body,
        grid=(TOK // 256,),
        in_specs=[blk, blk],
        out_specs=blk,
        out_shape=jax.ShapeDtypeStruct((TOK, OUT_DIM), jnp.float32),
    )(ya, yb)


def kernel(id_emb, llm_emb, W1, b1, W2, b2, Wg1, bg1, Wg2, bg2):
    combined = jnp.concatenate([id_emb, llm_emb], axis=-1)

    d1i, d2i, p1, p2, be2d, ba2d = _gate_route(combined, Wg1, bg1, Wg2, bg2)

    rtok2d, rwt2d = _invert(d1i.astype(jnp.float32), d2i.astype(jnp.float32),
                            p1, p2)
    rtok = rtok2d.reshape(R)
    rwt = rwt2d.reshape(R)

    xs = _gather_rows(rtok, combined)

    y = _grouped_mlp(be2d.reshape(NBP), ba2d.reshape(NBP), xs,
                     W1, b1, W2, b2, rwt.reshape(NB, BR, 1))

    ya, yb = _combine(d1i.reshape(TOK), d2i.reshape(TOK), y)
    return _pairwise_add(ya, yb)


# speedup vs baseline: 1.0038x; 1.0038x over previous
import jax
import jax.numpy as jnp
from jax import lax
from jax.experimental import pallas as pl
from jax.experimental.pallas import tpu as pltpu
from jax.experimental.pallas import tpu_sc as plsc

ID_DIM = 128
LLM_DIM = 2048
OUT_DIM = 1024
E = 8
TOK = 2048
IN_DIM = ID_DIM + LLM_DIM
HID = 2 * OUT_DIM

NC = 2
NS = 16
L = 16

BR = 128
NB = TOK * 2 // BR + E
NBP = 48
R = NB * BR
RB = 1024
NBR = R // RB

_SC_MESH = dict(core_axis_name="c", subcore_axis_name="s",
                num_cores=NC, num_subcores=NS)


def _gate_body(x_ref, wg1_ref, bg1_ref, wg2_ref, bg2_ref,
               d1_ref, d2_ref, p1_ref, p2_ref, be_ref, ba_ref):
    x = x_ref[...]
    h = jnp.maximum(
        jnp.dot(x, wg1_ref[...], preferred_element_type=jnp.float32)
        + bg1_ref[...], 0.0)
    logits = (jnp.dot(h, wg2_ref[...], preferred_element_type=jnp.float32)
              + bg2_ref[...])
    iota_e = lax.broadcasted_iota(jnp.int32, (TOK, E), 1)
    m1 = jnp.max(logits, axis=1, keepdims=True)
    idx1 = jnp.min(jnp.where(logits == m1, iota_e, E), axis=1, keepdims=True)
    masked = jnp.where(iota_e == idx1, -jnp.inf, logits)
    m2 = jnp.max(masked, axis=1, keepdims=True)
    idx2 = jnp.min(jnp.where(masked == m2, iota_e, E), axis=1, keepdims=True)
    p2 = 1.0 / (1.0 + jnp.exp(m1 - m2))
    p1_ref[...] = 1.0 - p2
    p2_ref[...] = p2

    oh1 = (iota_e == idx1).astype(jnp.float32)
    oh2 = (iota_e == idx2).astype(jnp.float32)
    cnt = oh1 + oh2
    rowi = lax.broadcasted_iota(jnp.int32, (TOK, TOK), 0)
    coli = lax.broadcasted_iota(jnp.int32, (TOK, TOK), 1)
    lt = (coli < rowi).astype(jnp.float32)
    prefix = jnp.dot(lt, cnt, preferred_element_type=jnp.float32)
    tot = jnp.sum(cnt, axis=0, keepdims=True)
    padded = jnp.ceil(tot / BR) * BR
    e_r = lax.broadcasted_iota(jnp.int32, (E, E), 0)
    e_c = lax.broadcasted_iota(jnp.int32, (E, E), 1)
    lt8 = (e_r < e_c).astype(jnp.float32)
    off = jnp.dot(padded, lt8, preferred_element_type=jnp.float32)
    dest1 = jnp.sum(oh1 * (off + prefix), axis=1, keepdims=True)
    dest2 = jnp.sum(oh2 * (off + prefix + oh1), axis=1, keepdims=True)
    d1_ref[...] = dest1.astype(jnp.int32)
    d2_ref[...] = dest2.astype(jnp.int32)

    blkoff = (off / BR).reshape(E, 1)
    nblk = (padded / BR).reshape(E, 1)
    totb = jnp.sum(padded / BR)
    bv = lax.broadcasted_iota(jnp.int32, (E, NBP), 1).astype(jnp.float32)
    ev = lax.broadcasted_iota(jnp.int32, (E, NBP), 0).astype(jnp.float32)
    cand = jnp.where((bv >= blkoff) & (nblk > 0), ev, 0.0)
    be_ref[...] = jnp.max(cand, axis=0, keepdims=True).astype(jnp.int32)
    bav = lax.broadcasted_iota(jnp.int32, (1, NBP), 1).astype(jnp.float32)
    ba_ref[...] = jnp.where(bav < totb, 1, 0).astype(jnp.int32)


def _gate_route(combined, Wg1, bg1, Wg2, bg2):
    return pl.pallas_call(
        _gate_body,
        out_shape=(
            jax.ShapeDtypeStruct((TOK, 1), jnp.int32),
            jax.ShapeDtypeStruct((TOK, 1), jnp.int32),
            jax.ShapeDtypeStruct((TOK, 1), jnp.float32),
            jax.ShapeDtypeStruct((TOK, 1), jnp.float32),
            jax.ShapeDtypeStruct((1, NBP), jnp.int32),
            jax.ShapeDtypeStruct((1, NBP), jnp.int32),
        ),
        compiler_params=pltpu.CompilerParams(
            vmem_limit_bytes=100 * 1024 * 1024),
    )(combined, Wg1, bg1, Wg2, bg2)


def _invert_body(d1_ref, d2_ref, w1_ref, w2_ref, rtok_ref, rwt_ref):
    i = pl.program_id(0)
    rbase = (i * RB).astype(jnp.float32)
    colv = (lax.broadcasted_iota(jnp.int32, (TOK, RB), 1).astype(jnp.float32)
            + rbase)
    m1 = (d1_ref[...] == colv).astype(jnp.float32)
    m2 = (d2_ref[...] == colv).astype(jnp.float32)
    tid = lax.broadcasted_iota(jnp.int32, (TOK, 1), 0).astype(jnp.float32)
    rtok = jnp.sum((m1 + m2) * tid, axis=0, keepdims=True)
    rwt = jnp.sum(m1 * w1_ref[...] + m2 * w2_ref[...], axis=0, keepdims=True)
    rtok_ref[...] = rtok.astype(jnp.int32).reshape(1, 1, RB)
    rwt_ref[...] = rwt.reshape(1, 1, RB)


def _invert(d1f, d2f, w1g, w2g):
    full = pl.BlockSpec((TOK, 1), lambda i: (0, 0))
    return pl.pallas_call(
        _invert_body,
        grid=(NBR,),
        in_specs=[full, full, full, full],
        out_specs=(pl.BlockSpec((1, 1, RB), lambda i: (i, 0, 0)),
                   pl.BlockSpec((1, 1, RB), lambda i: (i, 0, 0))),
        out_shape=(jax.ShapeDtypeStruct((NBR, 1, RB), jnp.int32),
                   jax.ShapeDtypeStruct((NBR, 1, RB), jnp.float32)),
        compiler_params=pltpu.CompilerParams(
            vmem_limit_bytes=100 * 1024 * 1024),
    )(d1f, d2f, w1g, w2g)


def _gather_body(tok_hbm, comb_hbm, xs_hbm, idx0, idx1, rows0, rows1,
                 sem0, sem1):
    cid = lax.axis_index("c")
    sid = lax.axis_index("s")
    wid = sid * NC + cid
    RPW = R // (NC * NS)
    CH = 16
    NCH = RPW // CH
    idxs = [idx0, idx1]
    rows = [rows0, rows1]
    sems = [sem0, sem1]
    pltpu.sync_copy(tok_hbm.at[pl.ds(wid * RPW, CH)], idx0)
    cps = [pltpu.async_copy(comb_hbm.at[idx0], rows0, sem0), None]
    for ch in range(NCH):
        cur, nxt = ch % 2, (ch + 1) % 2
        if ch + 1 < NCH:
            nbase = wid * RPW + (ch + 1) * CH
            pltpu.sync_copy(tok_hbm.at[pl.ds(nbase, CH)], idxs[nxt])
            cps[nxt] = pltpu.async_copy(comb_hbm.at[idxs[nxt]], rows[nxt],
                                        sems[nxt])
        cps[cur].wait()
        pltpu.sync_copy(rows[cur], xs_hbm.at[pl.ds(wid * RPW + ch * CH, CH)])


def _gather_rows(row_token, combined):
    return pl.kernel(
        _gather_body,
        out_type=jax.ShapeDtypeStruct((R, IN_DIM), jnp.float32),
        mesh=plsc.VectorSubcoreMesh(**_SC_MESH),
        scratch_types=[
            pltpu.VMEM((16,), jnp.int32),
            pltpu.VMEM((16,), jnp.int32),
            pltpu.VMEM((16, IN_DIM), jnp.float32),
            pltpu.VMEM((16, IN_DIM), jnp.float32),
            pltpu.SemaphoreType.DMA,
            pltpu.SemaphoreType.DMA,
        ],
    )(row_token, combined)


def _mlp_body(be_ref, ba_ref, xs_ref, w1_ref, b1_ref, w2_ref, b2_ref,
              wrow_ref, y_ref):
    i = pl.program_id(0)

    @pl.when(ba_ref[i] == 1)
    def _compute():
        h = jnp.maximum(
            jnp.dot(xs_ref[...], w1_ref[0],
                    preferred_element_type=jnp.float32) + b1_ref[0], 0.0)
        y = (jnp.dot(h, w2_ref[0], preferred_element_type=jnp.float32)
             + b2_ref[0])
        y_ref[...] = y * wrow_ref[0]


def _grouped_mlp(blk_e, blk_a, xs, W1, b1, W2, b2, wrow):
    grid_spec = pltpu.PrefetchScalarGridSpec(
        num_scalar_prefetch=2,
        grid=(NB,),
        in_specs=[
            pl.BlockSpec((BR, IN_DIM),
                         lambda i, be, ba: (jnp.where(ba[i] == 1, i, 0), 0)),
            pl.BlockSpec((1, IN_DIM, HID), lambda i, be, ba: (be[i], 0, 0)),
            pl.BlockSpec((1, 1, HID), lambda i, be, ba: (be[i], 0, 0)),
            pl.BlockSpec((1, HID, OUT_DIM), lambda i, be, ba: (be[i], 0, 0)),
            pl.BlockSpec((1, 1, OUT_DIM), lambda i, be, ba: (be[i], 0, 0)),
            pl.BlockSpec((1, BR, 1), lambda i, be, ba: (i, 0, 0)),
        ],
        out_specs=pl.BlockSpec((BR, OUT_DIM), lambda i, be, ba: (i, 0)),
    )
    return pl.pallas_call(
        _mlp_body,
        grid_spec=grid_spec,
        out_shape=jax.ShapeDtypeStruct((R, OUT_DIM), jnp.float32),
        compiler_params=pltpu.CompilerParams(
            vmem_limit_bytes=100 * 1024 * 1024),
    )(blk_e, blk_a, xs, W1, b1.reshape(E, 1, HID), W2,
      b2.reshape(E, 1, OUT_DIM), wrow)


def _combine_body(d1_hbm, d2_hbm, y_hbm, ya_hbm, yb_hbm, i0, i1, r0, r1, sem):
    cid = lax.axis_index("c")
    sid = lax.axis_index("s")
    wid = sid * NC + cid
    TPW = TOK // (NC * NS)
    CH = 32
    for ch in range(TPW // CH):
        base = wid * TPW + ch * CH
        pltpu.sync_copy(d1_hbm.at[pl.ds(base, CH)], i0)
        pltpu.sync_copy(d2_hbm.at[pl.ds(base, CH)], i1)
        pltpu.async_copy(y_hbm.at[i0], r0, sem).wait()
        pltpu.sync_copy(r0, ya_hbm.at[pl.ds(base, CH)])
        pltpu.async_copy(y_hbm.at[i1], r1, sem).wait()
        pltpu.sync_copy(r1, yb_hbm.at[pl.ds(base, CH)])


def _combine(d1, d2, y):
    return pl.kernel(
        _combine_body,
        out_type=(jax.ShapeDtypeStruct((TOK, OUT_DIM), jnp.float32),
                  jax.ShapeDtypeStruct((TOK, OUT_DIM), jnp.float32)),
        mesh=plsc.VectorSubcoreMesh(**_SC_MESH),
        scratch_types=[
            pltpu.VMEM((32,), jnp.int32),
            pltpu.VMEM((32,), jnp.int32),
            pltpu.VMEM((32, OUT_DIM), jnp.float32),
            pltpu.VMEM((32, OUT_DIM), jnp.float32),
            pltpu.SemaphoreType.DMA,
        ],
    )(d1, d2, y)


def _add_body(a_ref, b_ref, o_ref):
    o_ref[...] = a_ref[...] + b_ref[...]


def _pairwise_add(ya, yb):
    blk = pl.BlockSpec((256, OUT_DIM), lambda i: (i, 0))
    return pl.pallas_call(
        _add_body,
        grid=(TOK // 256,),
        in_specs=[blk, blk],
        out_specs=blk,
        out_shape=jax.ShapeDtypeStruct((TOK, OUT_DIM), jnp.float32),
    )(ya, yb)


def kernel(id_emb, llm_emb, W1, b1, W2, b2, Wg1, bg1, Wg2, bg2):
    combined = jnp.concatenate([id_emb, llm_emb], axis=-1)

    d1i, d2i, p1, p2, be2d, ba2d = _gate_route(combined, Wg1, bg1, Wg2, bg2)

    rtok2d, rwt2d = _invert(d1i.astype(jnp.float32), d2i.astype(jnp.float32),
                            p1, p2)
    rtok = rtok2d.reshape(R)
    rwt = rwt2d.reshape(R)

    xs = _gather_rows(rtok, combined)

    y = _grouped_mlp(be2d.reshape(NBP), ba2d.reshape(NBP), xs,
                     W1, b1, W2, b2, rwt.reshape(NB, BR, 1))

    ya, yb = _combine(d1i.reshape(TOK), d2i.reshape(TOK), y)
    return _pairwise_add(ya, yb)
